# baseline (speedup 1.0000x reference)
import functools

import jax
import jax.numpy as jnp
from jax import lax
from jax.experimental import pallas as pl
from jax.experimental.pallas import tpu as pltpu
from jax.experimental.pallas import tpu_sc as plsc

_BATCH = 16384
_NUM_COLS = 26
_ROWS_PER_TABLE = 100001
_D = 32
_TOTAL = _BATCH * _NUM_COLS

_NC = 2
_NS = 16
_NW = _NC * _NS
_PER_W = _TOTAL // _NW
_CH = 1024
_NCHUNK = _PER_W // _CH
_GPC = _CH // 128
_LANES = 16


def _flat_gather(xf, tf):
    mesh = plsc.VectorSubcoreMesh(core_axis_name="c", subcore_axis_name="s")

    @functools.partial(
        pl.kernel,
        mesh=mesh,
        compiler_params=pltpu.CompilerParams(use_tc_tiling_on_sc=False),
        out_type=jax.ShapeDtypeStruct((_TOTAL, _D), jnp.float32),
        scratch_types=[
            pltpu.VMEM((_CH,), jnp.int32),
            pltpu.VMEM((_GPC, 128), jnp.int32),
            pltpu.VMEM((_CH, _D), jnp.float32),
            pltpu.SemaphoreType.DMA,
        ],
    )
    def run(x_hbm, tbl_hbm, out_hbm, xb, fidx, rows, sem):
        wid = lax.axis_index("s") * _NC + lax.axis_index("c")
        base = wid * _PER_W

        def chunk(g, carry):
            off = base + g * _CH
            pltpu.sync_copy(x_hbm.at[pl.ds(off, _CH)], xb)
            for j in range(_GPC):
                for k in range(128 // _LANES):
                    p = j * 128 + k * _LANES
                    col = lax.rem(
                        off + p + lax.iota(jnp.int32, _LANES), _NUM_COLS)
                    fidx[j, pl.ds(k * _LANES, _LANES)] = (
                        xb[pl.ds(p, _LANES)] + col * _ROWS_PER_TABLE + 1)
            copies = [
                pltpu.async_copy(
                    tbl_hbm.at[fidx.at[j]],
                    rows.at[pl.ds(j * 128, 128)],
                    sem,
                )
                for j in range(_GPC)
            ]
            for c in copies:
                c.wait()
            pltpu.sync_copy(rows, out_hbm.at[pl.ds(off, _CH)])
            return carry

        lax.fori_loop(0, _NCHUNK, chunk, 0)

    return run(xf, tf)


def kernel(x, tables):
    xf = x.reshape(_TOTAL)
    tf = tables.reshape(_NUM_COLS * _ROWS_PER_TABLE, _D)
    out = _flat_gather(xf, tf)
    return out.reshape(_BATCH, _NUM_COLS, _D)


# speedup vs baseline: 8.8063x; 8.8063x over previous
import functools

import jax
import jax.numpy as jnp
from jax import lax
from jax.experimental import pallas as pl
from jax.experimental.pallas import tpu as pltpu
from jax.experimental.pallas import tpu_sc as plsc

_BATCH = 16384
_NUM_COLS = 26
_D = 32

_NC = 2
_NS = 16
_NW = _NC * _NS
_BPW = _BATCH // _NW
_IPW = _BPW * _NUM_COLS
_CHB = 16
_NCH = _BPW // _CHB


def kernel(x, tables):
    mesh = plsc.VectorSubcoreMesh(core_axis_name="c", subcore_axis_name="s")

    @functools.partial(
        pl.kernel,
        mesh=mesh,
        out_type=jax.ShapeDtypeStruct((_BATCH, _NUM_COLS, _D), jnp.float32),
        scratch_types=[
            pltpu.VMEM((_IPW,), jnp.int32),
            pltpu.VMEM((_CHB, _NUM_COLS, _D), jnp.float32),
            pltpu.SemaphoreType.DMA,
        ],
    )
    def run(x_hbm, tbl_hbm, out_hbm, xb, rows, sem):
        wid = lax.axis_index("s") * _NC + lax.axis_index("c")
        row0 = wid * _BPW
        pltpu.sync_copy(x_hbm.at[pl.ds(row0 * _NUM_COLS, _IPW)], xb)

        def chunk(t, carry):
            def bloop(bb, c2):
                p = (t * _CHB + bb) * _NUM_COLS
                lo = xb[pl.ds(p, 16)]
                hi = xb[pl.ds(p + _NUM_COLS - 16, 16)]
                copies = []
                for c in range(_NUM_COLS):
                    idx = (lo[c] if c < 16 else hi[c - (_NUM_COLS - 16)]) + 1
                    copies.append(pltpu.async_copy(
                        tbl_hbm.at[c, idx], rows.at[bb, c], sem))
                for cp in copies:
                    cp.wait()
                return c2

            lax.fori_loop(0, _CHB, bloop, 0)
            pltpu.sync_copy(rows, out_hbm.at[pl.ds(row0 + t * _CHB, _CHB)])
            return carry

        lax.fori_loop(0, _NCH, chunk, 0)

    return run(x.reshape(_BATCH * _NUM_COLS), tables)


# speedup vs baseline: 10.8236x; 1.2291x over previous
import functools

import jax
import jax.numpy as jnp
from jax import lax
from jax.experimental import pallas as pl
from jax.experimental.pallas import tpu as pltpu
from jax.experimental.pallas import tpu_sc as plsc

_BATCH = 16384
_NUM_COLS = 26
_D = 32

_NC = 2
_NS = 16
_NW = _NC * _NS
_BPW = _BATCH // _NW
_IPW = _BPW * _NUM_COLS
_CHB = 8
_NCH = _BPW // _CHB


def kernel(x, tables):
    mesh = plsc.VectorSubcoreMesh(core_axis_name="c", subcore_axis_name="s")

    @functools.partial(
        pl.kernel,
        mesh=mesh,
        out_type=jax.ShapeDtypeStruct((_BATCH, _NUM_COLS, _D), jnp.float32),
        scratch_types=[
            pltpu.VMEM((_IPW,), jnp.int32),
            pltpu.VMEM((_CHB, _NUM_COLS, _D), jnp.float32),
            pltpu.SemaphoreType.DMA,
        ],
    )
    def run(x_hbm, tbl_hbm, out_hbm, xb, rows, sem):
        wid = lax.axis_index("s") * _NC + lax.axis_index("c")
        row0 = wid * _BPW
        pltpu.sync_copy(x_hbm.at[pl.ds(row0 * _NUM_COLS, _IPW)], xb)

        def chunk(t, carry):
            copies = []
            for bb in range(_CHB):
                p = (t * _CHB + bb) * _NUM_COLS
                lo = xb[pl.ds(p, 16)]
                hi = xb[pl.ds(p + _NUM_COLS - 16, 16)]
                for c in range(_NUM_COLS):
                    idx = (lo[c] if c < 16 else hi[c - (_NUM_COLS - 16)]) + 1
                    copies.append(pltpu.async_copy(
                        tbl_hbm.at[c, idx], rows.at[bb, c], sem))
            for cp in copies:
                cp.wait()
            pltpu.sync_copy(rows, out_hbm.at[pl.ds(row0 + t * _CHB, _CHB)])
            return carry

        lax.fori_loop(0, _NCH, chunk, 0)

    return run(x.reshape(_BATCH * _NUM_COLS), tables)
